# Initial kernel scaffold; baseline (speedup 1.0000x reference)
import jax
import jax.numpy as jnp
from jax.experimental import pallas as pl


def kernel(x_ids, emb, wc, wl, wo):
    raise NotImplementedError("write your pallas kernel here")


# speedup vs baseline: 1.3910x; 1.3910x over previous
import jax
import jax.numpy as jnp
from jax.experimental import pallas as pl
from jax.experimental.pallas import tpu as pltpu

_EMB = 16
_WIN = 36
_TF = 4
_CS = 48
_LIN = 192
_NF = 8
_NA = 5
_KIN = 320
_XW = _WIN * _EMB
_MAX_BB = 512


def _make_body(bb):
    def _body(xe_ref, w2_ref, bc_ref, wl_ref, wo_ref, out_ref):
        xe = xe_ref[...]
        pooled = []
        for t in range(_TF):
            xt = xe[:, 128 * t:128 * t + 192]
            conv = jnp.dot(xt, w2_ref[...],
                           preferred_element_type=jnp.float32)
            m = jnp.maximum(conv[:, :192], conv[:, 192:])
            m = jnp.maximum(m[:, :96], m[:, 96:])
            m = jnp.maximum(m[:, :48], m[:, 48:])
            pooled.append(m)
        feats = jnp.concatenate(pooled, axis=1)
        feats = jnp.maximum(feats + bc_ref[0:1, :], 0.0)

        h = jnp.maximum(
            jnp.dot(feats.astype(jnp.bfloat16), wl_ref[0:_LIN, :],
                    preferred_element_type=jnp.float32)
            + wl_ref[_LIN:_LIN + 1, :].astype(jnp.float32), 0.0)

        o = jnp.maximum(
            jnp.dot(h.astype(jnp.bfloat16), wo_ref[0:_NF, :],
                    preferred_element_type=jnp.float32)
            + wo_ref[_NF:_NF + 1, :].astype(jnp.float32), 0.0)

        mx = jnp.max(o, axis=-1, keepdims=True)
        lse = mx + jnp.log(jnp.sum(jnp.exp(o - mx), axis=-1, keepdims=True))
        out_ref[...] = o - lse
    return _body


@jax.jit
def kernel(x_ids, emb, wc, wl, wo):
    B = x_ids.shape[0]

    w80 = wc[0:80, 0:_CS]
    w2 = jnp.concatenate(
        [jnp.pad(w80, ((16 * r, 112 - 16 * r), (0, 0))) for r in range(8)],
        axis=1)

    bconv = jnp.broadcast_to(wc[_KIN, 0:_LIN].astype(jnp.float32),
                             (8, _LIN))

    xe = jnp.take(emb, x_ids, axis=0).reshape(B, _XW)

    bb = B if B <= _MAX_BB else _MAX_BB
    Bp = ((B + bb - 1) // bb) * bb
    if Bp != B:
        xe = jnp.pad(xe, ((0, Bp - B), (0, 0)))

    const2 = lambda i: (0, 0)
    out = pl.pallas_call(
        _make_body(bb),
        out_shape=jax.ShapeDtypeStruct((Bp, _NA), jnp.float32),
        grid=(Bp // bb,),
        in_specs=[
            pl.BlockSpec((bb, _XW), lambda i: (i, 0)),
            pl.BlockSpec((_LIN, 384), const2),
            pl.BlockSpec((8, _LIN), const2),
            pl.BlockSpec((_LIN + 1, _NF), const2),
            pl.BlockSpec((_NF + 1, _NA), const2),
        ],
        out_specs=pl.BlockSpec((bb, _NA), lambda i: (i, 0)),
        compiler_params=pltpu.CompilerParams(
            dimension_semantics=("parallel",),
            allow_input_fusion=[True, False, False, False, False],
        ),
    )(xe, w2, bconv, wl, wo)

    return out[:B]


# speedup vs baseline: 37.4440x; 26.9188x over previous
import jax
import jax.numpy as jnp
from jax.experimental import pallas as pl
from jax.experimental.pallas import tpu as pltpu

_EMB = 16
_WIN = 36
_VP = 64
_TF = 4
_CS = 48
_LIN = 192
_NF = 8
_NA = 5
_KIN = 320
_KREP = _WIN * _VP
_MAX_BB = 512


def _make_body(bb):
    def _body(x_ref, r_ref, vpat_ref, wcomb_ref, bc_ref, wl_ref, wo_ref,
              out_ref):
        xf = x_ref[...].astype(jnp.bfloat16)
        xrep = jnp.dot(xf, r_ref[...],
                       preferred_element_type=jnp.float32)
        oh = (xrep.astype(jnp.bfloat16) == vpat_ref[0:1, :]
              ).astype(jnp.bfloat16)

        pooled = []
        for t in range(_TF):
            conv = jnp.dot(oh[:, 512 * t:512 * t + 768], wcomb_ref[...],
                           preferred_element_type=jnp.float32)
            m = jnp.maximum(conv[:, :192], conv[:, 192:])
            m = jnp.maximum(m[:, :96], m[:, 96:])
            m = jnp.maximum(m[:, :48], m[:, 48:])
            pooled.append(m)
        feats = jnp.concatenate(pooled, axis=1)
        feats = jnp.maximum(feats + bc_ref[0:1, :], 0.0)

        h = jnp.maximum(
            jnp.dot(feats.astype(jnp.bfloat16), wl_ref[0:_LIN, :],
                    preferred_element_type=jnp.float32)
            + wl_ref[_LIN:_LIN + 1, :].astype(jnp.float32), 0.0)

        o = jnp.maximum(
            jnp.dot(h.astype(jnp.bfloat16), wo_ref[0:_NF, :],
                    preferred_element_type=jnp.float32)
            + wo_ref[_NF:_NF + 1, :].astype(jnp.float32), 0.0)

        mx = jnp.max(o, axis=-1, keepdims=True)
        lse = mx + jnp.log(jnp.sum(jnp.exp(o - mx), axis=-1, keepdims=True))
        out_ref[...] = o - lse
    return _body


@jax.jit
def kernel(x_ids, emb, wc, wl, wo):
    B = x_ids.shape[0]
    f32 = jnp.float32

    emb64 = jnp.pad(emb.astype(f32), ((0, _VP - emb.shape[0]), (0, 0)))
    w80 = wc[0:80, 0:_CS].astype(f32).reshape(5, _EMB, _CS)
    tt = jnp.einsum('ve,mec->mvc', emb64, w80).reshape(320, _CS)
    wcomb = jnp.concatenate(
        [jnp.pad(tt, ((64 * r, 448 - 64 * r), (0, 0))) for r in range(8)],
        axis=1).astype(jnp.bfloat16)

    rep = jnp.kron(jnp.eye(_WIN, dtype=f32),
                   jnp.ones((1, _VP), f32)).astype(jnp.bfloat16)
    vpat = jnp.broadcast_to(
        jnp.tile(jnp.arange(_VP, dtype=f32), _WIN)[None, :],
        (8, _KREP)).astype(jnp.bfloat16)

    bconv = jnp.broadcast_to(wc[_KIN, 0:_LIN].astype(f32), (8, _LIN))

    bb = B if B <= _MAX_BB else _MAX_BB
    Bp = ((B + bb - 1) // bb) * bb
    if Bp != B:
        x_ids = jnp.pad(x_ids, ((0, Bp - B), (0, 0)))

    const2 = lambda i: (0, 0)
    out = pl.pallas_call(
        _make_body(bb),
        out_shape=jax.ShapeDtypeStruct((Bp, _NA), jnp.float32),
        grid=(Bp // bb,),
        in_specs=[
            pl.BlockSpec((bb, _WIN), lambda i: (i, 0)),
            pl.BlockSpec((_WIN, _KREP), const2),
            pl.BlockSpec((8, _KREP), const2),
            pl.BlockSpec((768, 384), const2),
            pl.BlockSpec((8, _LIN), const2),
            pl.BlockSpec((_LIN + 1, _NF), const2),
            pl.BlockSpec((_NF + 1, _NA), const2),
        ],
        out_specs=pl.BlockSpec((bb, _NA), lambda i: (i, 0)),
        compiler_params=pltpu.CompilerParams(
            dimension_semantics=("parallel",),
        ),
    )(x_ids, rep, vpat, wcomb, bconv, wl, wo)

    return out[:B]


# speedup vs baseline: 37.5450x; 1.0027x over previous
import jax
import jax.numpy as jnp
from jax.experimental import pallas as pl
from jax.experimental.pallas import tpu as pltpu

_EMB = 16
_WIN = 36
_VP = 64
_TF = 4
_CS = 48
_CP = 64
_LIN = 192
_NF = 8
_NA = 5
_KIN = 320
_KREP = _WIN * _VP
_MAX_BB = 1024


def _make_body(bb):
    def _body(x_ref, wcomb_ref, bc_ref, wl_ref, wo_ref, out_ref):
        xb = x_ref[...].astype(jnp.bfloat16)
        v64 = jax.lax.broadcasted_iota(
            jnp.int32, (1, _VP), 1).astype(jnp.bfloat16)
        oh = jnp.concatenate(
            [(xb[:, j:j + 1] == v64).astype(jnp.bfloat16)
             for j in range(_WIN)], axis=1)

        pooled = []
        for t in range(_TF):
            conv = jnp.dot(oh[:, 512 * t:512 * t + 768], wcomb_ref[...],
                           preferred_element_type=jnp.float32)
            m = jnp.maximum(conv[:, :256], conv[:, 256:])
            m = jnp.maximum(m[:, :128], m[:, 128:])
            m = jnp.maximum(m[:, :64], m[:, 64:])
            pooled.append(m)
        feats = jnp.concatenate(pooled, axis=1)
        feats = jnp.maximum(feats + bc_ref[0:1, :], 0.0)

        h = jnp.maximum(
            jnp.dot(feats.astype(jnp.bfloat16), wl_ref[0:256, :],
                    preferred_element_type=jnp.float32)
            + wl_ref[256:257, :].astype(jnp.float32), 0.0)

        o = jnp.maximum(
            jnp.dot(h.astype(jnp.bfloat16), wo_ref[0:_NF, :],
                    preferred_element_type=jnp.float32)
            + wo_ref[_NF:_NF + 1, :].astype(jnp.float32), 0.0)

        mx = jnp.max(o, axis=-1, keepdims=True)
        lse = mx + jnp.log(jnp.sum(jnp.exp(o - mx), axis=-1, keepdims=True))
        out_ref[...] = o - lse
    return _body


@jax.jit
def kernel(x_ids, emb, wc, wl, wo):
    B = x_ids.shape[0]
    f32 = jnp.float32

    emb64 = jnp.pad(emb.astype(f32), ((0, _VP - emb.shape[0]), (0, 0)))
    w80 = wc[0:80, 0:_CS].astype(f32).reshape(5, _EMB, _CS)
    tt = jnp.einsum('ve,mec->mvc', emb64, w80).reshape(320, _CS)
    tt = jnp.pad(tt, ((0, 0), (0, _CP - _CS)))
    wcomb = jnp.concatenate(
        [jnp.pad(tt, ((64 * r, 448 - 64 * r), (0, 0))) for r in range(8)],
        axis=1).astype(jnp.bfloat16)

    b256 = jnp.pad(wc[_KIN, 0:_LIN].astype(f32).reshape(_TF, _CS),
                   ((0, 0), (0, _CP - _CS))).reshape(_TF * _CP)
    bconv = jnp.broadcast_to(b256, (8, _TF * _CP))

    wl2 = jnp.pad(wl[0:_LIN, :].reshape(_TF, _CS, _NF),
                  ((0, 0), (0, _CP - _CS), (0, 0))).reshape(_TF * _CP, _NF)
    wl2 = jnp.concatenate([wl2, wl[_LIN:_LIN + 1, :]], axis=0)

    bb = B if B <= _MAX_BB else _MAX_BB
    Bp = ((B + bb - 1) // bb) * bb
    if Bp != B:
        x_ids = jnp.pad(x_ids, ((0, Bp - B), (0, 0)))

    const2 = lambda i: (0, 0)
    out = pl.pallas_call(
        _make_body(bb),
        out_shape=jax.ShapeDtypeStruct((Bp, _NA), jnp.float32),
        grid=(Bp // bb,),
        in_specs=[
            pl.BlockSpec((bb, _WIN), lambda i: (i, 0)),
            pl.BlockSpec((768, 512), const2),
            pl.BlockSpec((8, _TF * _CP), const2),
            pl.BlockSpec((257, _NF), const2),
            pl.BlockSpec((_NF + 1, _NA), const2),
        ],
        out_specs=pl.BlockSpec((bb, _NA), lambda i: (i, 0)),
        compiler_params=pltpu.CompilerParams(
            dimension_semantics=("parallel",),
        ),
    )(x_ids, wcomb, bconv, wl2, wo)

    return out[:B]


# speedup vs baseline: 52.2440x; 1.3915x over previous
import jax
import jax.numpy as jnp
from jax.experimental import pallas as pl
from jax.experimental.pallas import tpu as pltpu

_EMB = 16
_WIN = 36
_VP = 64
_TF = 4
_CS = 48
_CP = 64
_LIN = 192
_NF = 8
_NA = 5
_KIN = 320
_KREP = _WIN * _VP
_MAX_BB = 1024


def _make_body(bb):
    def _body(x_ref, raug_ref, wcomb_ref, bc_ref, wl_ref, wo_ref, out_ref):
        xb = x_ref[...].astype(jnp.bfloat16)
        xaug = jnp.concatenate(
            [xb, jnp.ones((bb, 1), jnp.bfloat16)], axis=1)
        d = jnp.dot(xaug, raug_ref[...],
                    preferred_element_type=jnp.float32)
        oh = (d.astype(jnp.bfloat16) == jnp.bfloat16(0.0)
              ).astype(jnp.bfloat16)

        pooled = []
        for t in range(_TF):
            s0 = 512 * t
            lo = jnp.dot(oh[:, s0:s0 + 512], wcomb_ref[...],
                         preferred_element_type=jnp.float32)
            hi = jnp.dot(oh[:, s0 + 256:s0 + 768], wcomb_ref[...],
                         preferred_element_type=jnp.float32)
            m = jnp.maximum(lo, hi)
            m = jnp.maximum(m[:, :128], m[:, 128:])
            m = jnp.maximum(m[:, :64], m[:, 64:])
            pooled.append(m)
        feats = jnp.concatenate(pooled, axis=1)
        feats = jnp.maximum(feats + bc_ref[0:1, :], 0.0)

        h = jnp.maximum(
            jnp.dot(feats.astype(jnp.bfloat16), wl_ref[0:256, :],
                    preferred_element_type=jnp.float32)
            + wl_ref[256:257, :].astype(jnp.float32), 0.0)

        o = jnp.maximum(
            jnp.dot(h.astype(jnp.bfloat16), wo_ref[0:_NF, :],
                    preferred_element_type=jnp.float32)
            + wo_ref[_NF:_NF + 1, :].astype(jnp.float32), 0.0)

        mx = jnp.max(o, axis=-1, keepdims=True)
        lse = mx + jnp.log(jnp.sum(jnp.exp(o - mx), axis=-1, keepdims=True))
        out_ref[...] = o - lse
    return _body


@jax.jit
def kernel(x_ids, emb, wc, wl, wo):
    B = x_ids.shape[0]
    f32 = jnp.float32

    emb64 = jnp.pad(emb.astype(f32), ((0, _VP - emb.shape[0]), (0, 0)))
    w80 = wc[0:80, 0:_CS].astype(f32).reshape(5, _EMB, _CS)
    tt = jnp.einsum('ve,mec->mvc', emb64, w80).reshape(320, _CS)
    tt = jnp.pad(tt, ((0, 0), (0, _CP - _CS)))
    wcomb = jnp.concatenate(
        [jnp.pad(tt, ((64 * r, 192 - 64 * r), (0, 0))) for r in range(4)],
        axis=1).astype(jnp.bfloat16)

    rep = jnp.kron(jnp.eye(_WIN, dtype=f32), jnp.ones((1, _VP), f32))
    vrow = -jnp.tile(jnp.arange(_VP, dtype=f32), _WIN)[None, :]
    raug = jnp.concatenate([rep, vrow], axis=0).astype(jnp.bfloat16)

    b256 = jnp.pad(wc[_KIN, 0:_LIN].astype(f32).reshape(_TF, _CS),
                   ((0, 0), (0, _CP - _CS))).reshape(_TF * _CP)
    bconv = jnp.broadcast_to(b256, (8, _TF * _CP))

    wl2 = jnp.pad(wl[0:_LIN, :].reshape(_TF, _CS, _NF),
                  ((0, 0), (0, _CP - _CS), (0, 0))).reshape(_TF * _CP, _NF)
    wl2 = jnp.concatenate([wl2, wl[_LIN:_LIN + 1, :]], axis=0)

    bb = B if B <= _MAX_BB else _MAX_BB
    Bp = ((B + bb - 1) // bb) * bb
    if Bp != B:
        x_ids = jnp.pad(x_ids, ((0, Bp - B), (0, 0)))

    const2 = lambda i: (0, 0)
    out = pl.pallas_call(
        _make_body(bb),
        out_shape=jax.ShapeDtypeStruct((Bp, _NA), jnp.float32),
        grid=(Bp // bb,),
        in_specs=[
            pl.BlockSpec((bb, _WIN), lambda i: (i, 0)),
            pl.BlockSpec((_WIN + 1, _KREP), const2),
            pl.BlockSpec((512, 256), const2),
            pl.BlockSpec((8, _TF * _CP), const2),
            pl.BlockSpec((257, _NF), const2),
            pl.BlockSpec((_NF + 1, _NA), const2),
        ],
        out_specs=pl.BlockSpec((bb, _NA), lambda i: (i, 0)),
        compiler_params=pltpu.CompilerParams(
            dimension_semantics=("parallel",),
        ),
    )(x_ids, raug, wcomb, bconv, wl2, wo)

    return out[:B]


# speedup vs baseline: 53.4247x; 1.0226x over previous
import jax
import jax.numpy as jnp
from jax.experimental import pallas as pl
from jax.experimental.pallas import tpu as pltpu

_EMB = 16
_WIN = 36
_VP = 64
_TF = 4
_CS = 48
_CP = 64
_LIN = 192
_NF = 8
_NA = 5
_KIN = 320
_KREP = _WIN * _VP
_MAX_BB = 2048


def _make_body(bb):
    def _body(x_ref, raug_ref, wcomb_ref, bc_ref, wl_ref, wo_ref, out_ref):
        xb = x_ref[...].astype(jnp.bfloat16)
        xaug = jnp.concatenate(
            [xb, jnp.ones((bb, 1), jnp.bfloat16)], axis=1)
        d = jnp.dot(xaug, raug_ref[...],
                    preferred_element_type=jnp.float32)
        oh = (d.astype(jnp.bfloat16) == jnp.bfloat16(0.0)
              ).astype(jnp.bfloat16)

        pooled = []
        for t in range(_TF):
            s0 = 512 * t
            lo = jnp.dot(oh[:, s0:s0 + 512], wcomb_ref[...],
                         preferred_element_type=jnp.float32)
            hi = jnp.dot(oh[:, s0 + 256:s0 + 768], wcomb_ref[...],
                         preferred_element_type=jnp.float32)
            m = jnp.maximum(lo, hi)
            m = jnp.maximum(m[:, :128], m[:, 128:])
            m = jnp.maximum(m[:, :64], m[:, 64:])
            pooled.append(m)
        feats = jnp.concatenate(pooled, axis=1)
        feats = jnp.maximum(feats + bc_ref[0:1, :], 0.0)

        h = jnp.maximum(
            jnp.dot(feats.astype(jnp.bfloat16), wl_ref[0:256, :],
                    preferred_element_type=jnp.float32)
            + wl_ref[256:257, :].astype(jnp.float32), 0.0)

        o = jnp.maximum(
            jnp.dot(h.astype(jnp.bfloat16), wo_ref[0:_NF, :],
                    preferred_element_type=jnp.float32)
            + wo_ref[_NF:_NF + 1, :].astype(jnp.float32), 0.0)

        mx = jnp.max(o, axis=-1, keepdims=True)
        lse = mx + jnp.log(jnp.sum(jnp.exp(o - mx), axis=-1, keepdims=True))
        out_ref[...] = o - lse
    return _body


@jax.jit
def kernel(x_ids, emb, wc, wl, wo):
    B = x_ids.shape[0]
    f32 = jnp.float32

    emb64 = jnp.pad(emb.astype(f32), ((0, _VP - emb.shape[0]), (0, 0)))
    w80 = wc[0:80, 0:_CS].astype(f32).reshape(5, _EMB, _CS)
    tt = jnp.einsum('ve,mec->mvc', emb64, w80).reshape(320, _CS)
    tt = jnp.pad(tt, ((0, 0), (0, _CP - _CS)))
    wcomb = jnp.concatenate(
        [jnp.pad(tt, ((64 * r, 192 - 64 * r), (0, 0))) for r in range(4)],
        axis=1).astype(jnp.bfloat16)

    rep = jnp.kron(jnp.eye(_WIN, dtype=f32), jnp.ones((1, _VP), f32))
    vrow = -jnp.tile(jnp.arange(_VP, dtype=f32), _WIN)[None, :]
    raug = jnp.concatenate([rep, vrow], axis=0).astype(jnp.bfloat16)

    b256 = jnp.pad(wc[_KIN, 0:_LIN].astype(f32).reshape(_TF, _CS),
                   ((0, 0), (0, _CP - _CS))).reshape(_TF * _CP)
    bconv = jnp.broadcast_to(b256, (8, _TF * _CP))

    wl2 = jnp.pad(wl[0:_LIN, :].reshape(_TF, _CS, _NF),
                  ((0, 0), (0, _CP - _CS), (0, 0))).reshape(_TF * _CP, _NF)
    wl2 = jnp.concatenate([wl2, wl[_LIN:_LIN + 1, :]], axis=0)

    bb = B if B <= _MAX_BB else _MAX_BB
    Bp = ((B + bb - 1) // bb) * bb
    if Bp != B:
        x_ids = jnp.pad(x_ids, ((0, Bp - B), (0, 0)))

    const2 = lambda i: (0, 0)
    out = pl.pallas_call(
        _make_body(bb),
        out_shape=jax.ShapeDtypeStruct((Bp, _NA), jnp.float32),
        grid=(Bp // bb,),
        in_specs=[
            pl.BlockSpec((bb, _WIN), lambda i: (i, 0)),
            pl.BlockSpec((_WIN + 1, _KREP), const2),
            pl.BlockSpec((512, 256), const2),
            pl.BlockSpec((8, _TF * _CP), const2),
            pl.BlockSpec((257, _NF), const2),
            pl.BlockSpec((_NF + 1, _NA), const2),
        ],
        out_specs=pl.BlockSpec((bb, _NA), lambda i: (i, 0)),
        compiler_params=pltpu.CompilerParams(
            dimension_semantics=("parallel",),
        ),
    )(x_ids, raug, wcomb, bconv, wl2, wo)

    return out[:B]


# speedup vs baseline: 60.6600x; 1.1354x over previous
import jax
import jax.numpy as jnp
from jax.experimental import pallas as pl
from jax.experimental.pallas import tpu as pltpu

_EMB = 16
_WIN = 36
_VP = 64
_TF = 4
_CS = 48
_CP = 64
_LIN = 192
_NF = 8
_NA = 5
_KIN = 320
_KREP = _WIN * _VP
_MAX_BB = 2048


def _make_body(bb):
    def _body(x_ref, raug_ref, wcomb_ref, bc_ref, wl_ref, wo_ref, out_ref):
        xb = x_ref[...].astype(jnp.bfloat16)
        xaug = jnp.concatenate(
            [xb, jnp.ones((bb, 1), jnp.bfloat16)], axis=1)
        d = jnp.dot(xaug, raug_ref[...],
                    preferred_element_type=jnp.float32)
        db = d.astype(jnp.bfloat16)
        oh = jnp.maximum(jnp.bfloat16(1.0) - jnp.abs(db),
                         jnp.bfloat16(0.0))

        pooled = []
        for t in range(_TF):
            s0 = 512 * t
            lo = jnp.dot(oh[:, s0:s0 + 512], wcomb_ref[...],
                         preferred_element_type=jnp.float32)
            hi = jnp.dot(oh[:, s0 + 256:s0 + 768], wcomb_ref[...],
                         preferred_element_type=jnp.float32)
            m = jnp.maximum(lo, hi)
            m = jnp.maximum(m[:, :128], m[:, 128:])
            m = jnp.maximum(m[:, :64], m[:, 64:])
            pooled.append(m)
        feats = jnp.concatenate(pooled, axis=1)
        feats = jnp.maximum(feats + bc_ref[0:1, :], 0.0)

        h = jnp.maximum(
            jnp.dot(feats.astype(jnp.bfloat16), wl_ref[0:256, :],
                    preferred_element_type=jnp.float32)
            + wl_ref[256:257, :].astype(jnp.float32), 0.0)

        o = jnp.maximum(
            jnp.dot(h.astype(jnp.bfloat16), wo_ref[0:_NF, :],
                    preferred_element_type=jnp.float32)
            + wo_ref[_NF:_NF + 1, :].astype(jnp.float32), 0.0)

        mx = jnp.max(o, axis=-1, keepdims=True)
        lse = mx + jnp.log(jnp.sum(jnp.exp(o - mx), axis=-1, keepdims=True))
        out_ref[...] = o - lse
    return _body


@jax.jit
def kernel(x_ids, emb, wc, wl, wo):
    B = x_ids.shape[0]
    f32 = jnp.float32

    emb64 = jnp.pad(emb.astype(f32), ((0, _VP - emb.shape[0]), (0, 0)))
    w80 = wc[0:80, 0:_CS].astype(f32).reshape(5, _EMB, _CS)
    tt = jnp.einsum('ve,mec->mvc', emb64, w80).reshape(320, _CS)
    tt = jnp.pad(tt, ((0, 0), (0, _CP - _CS)))
    wcomb = jnp.concatenate(
        [jnp.pad(tt, ((64 * r, 192 - 64 * r), (0, 0))) for r in range(4)],
        axis=1).astype(jnp.bfloat16)

    rep = jnp.kron(jnp.eye(_WIN, dtype=f32), jnp.ones((1, _VP), f32))
    vrow = -jnp.tile(jnp.arange(_VP, dtype=f32), _WIN)[None, :]
    raug = jnp.concatenate([rep, vrow], axis=0).astype(jnp.bfloat16)

    b256 = jnp.pad(wc[_KIN, 0:_LIN].astype(f32).reshape(_TF, _CS),
                   ((0, 0), (0, _CP - _CS))).reshape(_TF * _CP)
    bconv = jnp.broadcast_to(b256, (8, _TF * _CP))

    wl2 = jnp.pad(wl[0:_LIN, :].reshape(_TF, _CS, _NF),
                  ((0, 0), (0, _CP - _CS), (0, 0))).reshape(_TF * _CP, _NF)
    wl2 = jnp.concatenate([wl2, wl[_LIN:_LIN + 1, :]], axis=0)

    bb = B if B <= _MAX_BB else _MAX_BB
    Bp = ((B + bb - 1) // bb) * bb
    if Bp != B:
        x_ids = jnp.pad(x_ids, ((0, Bp - B), (0, 0)))

    const2 = lambda i: (0, 0)
    out = pl.pallas_call(
        _make_body(bb),
        out_shape=jax.ShapeDtypeStruct((Bp, _NA), jnp.float32),
        grid=(Bp // bb,),
        in_specs=[
            pl.BlockSpec((bb, _WIN), lambda i: (i, 0)),
            pl.BlockSpec((_WIN + 1, _KREP), const2),
            pl.BlockSpec((512, 256), const2),
            pl.BlockSpec((8, _TF * _CP), const2),
            pl.BlockSpec((257, _NF), const2),
            pl.BlockSpec((_NF + 1, _NA), const2),
        ],
        out_specs=pl.BlockSpec((bb, _NA), lambda i: (i, 0)),
        compiler_params=pltpu.CompilerParams(
            dimension_semantics=("parallel",),
        ),
    )(x_ids, raug, wcomb, bconv, wl2, wo)

    return out[:B]
